# baseline (speedup 1.0000x reference)
import functools

import jax
import jax.numpy as jnp
from jax import lax
from jax.experimental import pallas as pl
from jax.experimental.pallas import tpu as pltpu

D_MODEL = 768
N_HEADS = 12
MAX_STATES = 64
ENERGY_BUDGET = 100.0
EPS = 1e-6


def _body(x_ref, wqkv_ref, wo_ref, be_ref,
          out_ref, q_ref, sk_ref, sv_ref, cnt_ref, cn_ref, acc_ref,
          *, base, ns, bs):
    t = pl.program_id(1)
    K = MAX_STATES
    H = N_HEADS
    dh = D_MODEL // H

    D = D_MODEL

    @pl.when(t < ns)
    def _phase0():
        xh = x_ref[0].astype(jnp.bfloat16)
        qkv = jnp.dot(xh, wqkv_ref[...], preferred_element_type=jnp.float32)
        q = qkv[:, :D]
        k = qkv[:, D:2 * D]
        v = qkv[:, 2 * D:3 * D]
        q_ref[pl.ds(t * bs, bs), :] = q.astype(jnp.bfloat16)

        dot = qkv[:, 3 * D:3 * D + 1] + be_ref[0, 0]
        w = base * 2.0 * jax.nn.sigmoid(dot)

        kn = k / (jnp.sqrt(jnp.sum(k * k, axis=-1, keepdims=True)) + EPS)

        @pl.when(t == 0)
        def _():
            cn_ref[...] = kn[:K]
            sk_ref[...] = jnp.zeros_like(sk_ref)
            sv_ref[...] = jnp.zeros_like(sv_ref)
            cnt_ref[...] = jnp.zeros_like(cnt_ref)

        sims = lax.dot_general(kn, cn_ref[...], (((1,), (1,)), ((), ())),
                               preferred_element_type=jnp.float32)
        m = jnp.max(sims, axis=-1, keepdims=True)
        j = lax.broadcasted_iota(jnp.int32, sims.shape, 1)
        assign = jnp.min(jnp.where(sims >= m, j, K), axis=-1, keepdims=True)
        p = (j == assign).astype(jnp.float32)

        sk_ref[...] += lax.dot_general(p, k * w, (((0,), (0,)), ((), ())),
                                       preferred_element_type=jnp.float32)
        sv_ref[...] += lax.dot_general(p, v * w, (((0,), (0,)), ((), ())),
                                       preferred_element_type=jnp.float32)
        cnt_ref[0:1, :] += jnp.sum(p * w, axis=0, keepdims=True)

    @pl.when(t >= ns)
    def _phase1():
        inv = 1.0 / (cnt_ref[0:1, :] + EPS)
        scale = inv / jnp.sqrt(jnp.float32(dh))
        q = q_ref[pl.ds((t - ns) * bs, bs), :]
        for h in range(H):
            lo = h * dh
            qh = q[:, lo:lo + dh]
            skh = sk_ref[:, lo:lo + dh].astype(jnp.bfloat16)
            svh = sv_ref[:, lo:lo + dh].astype(jnp.bfloat16)
            scores = lax.dot_general(qh, skh, (((1,), (1,)), ((), ())),
                                     preferred_element_type=jnp.float32) * scale
            mx = jnp.max(scores, axis=-1, keepdims=True)
            e = jnp.exp(scores - mx)
            attn = (e / jnp.sum(e, axis=-1, keepdims=True)) * inv
            oh = jnp.dot(attn.astype(jnp.bfloat16), svh,
                         preferred_element_type=jnp.float32)
            acc_ref[:, lo:lo + dh] = oh.astype(jnp.bfloat16)
        out_ref[0] = jnp.dot(acc_ref[...], wo_ref[...],
                             preferred_element_type=jnp.float32)


def kernel(x, W_q, W_k, W_v, W_o, w_e, b_e):
    B, S, D = x.shape
    K = MAX_STATES
    BS = 1024 if S % 1024 == 0 else (512 if S % 512 == 0 else S)
    ns = S // BS
    base = ENERGY_BUDGET / S

    be_11 = b_e.reshape(1, 1)
    DP = 3 * D + 128
    wqkv = jnp.concatenate(
        [W_q, W_k, W_v,
         jnp.pad(w_e, ((0, 0), (0, 127)))], axis=1).astype(jnp.bfloat16)
    wo_h = W_o.astype(jnp.bfloat16)

    full = lambda b, t: (0, 0)
    out = pl.pallas_call(
        functools.partial(_body, base=base, ns=ns, bs=BS),
        grid=(B, 2 * ns),
        in_specs=[
            pl.BlockSpec((1, BS, D),
                         lambda b, t: (b, jnp.minimum(t, ns - 1), 0)),
            pl.BlockSpec((D, DP), full),
            pl.BlockSpec((D, D), full),
            pl.BlockSpec((1, 1), full),
        ],
        out_specs=pl.BlockSpec((1, BS, D),
                               lambda b, t: (b, jnp.maximum(t - ns, 0), 0)),
        out_shape=jax.ShapeDtypeStruct((B, S, D), jnp.float32),
        scratch_shapes=[
            pltpu.VMEM((S, D), jnp.bfloat16),
            pltpu.VMEM((K, D), jnp.float32),
            pltpu.VMEM((K, D), jnp.float32),
            pltpu.VMEM((8, K), jnp.float32),
            pltpu.VMEM((K, D), jnp.float32),
            pltpu.VMEM((BS, D), jnp.bfloat16),
        ],
        compiler_params=pltpu.CompilerParams(
            dimension_semantics=("arbitrary", "arbitrary")),
    )(x, wqkv, wo_h, be_11)
    return out


# speedup vs baseline: 1.0200x; 1.0200x over previous
import functools

import jax
import jax.numpy as jnp
from jax import lax
from jax.experimental import pallas as pl
from jax.experimental.pallas import tpu as pltpu

D_MODEL = 768
N_HEADS = 12
MAX_STATES = 64
ENERGY_BUDGET = 100.0
EPS = 1e-6


def _body(x_ref, wq_ref, wk_ref, wv_ref, wo_ref, we_ref, be_ref,
          out_ref, q_ref, sk_ref, sv_ref, cnt_ref, cn_ref, acc_ref,
          *, base, ns, bs):
    t = pl.program_id(1)
    K = MAX_STATES
    H = N_HEADS
    dh = D_MODEL // H

    @pl.when(t < ns)
    def _phase0():
        x = x_ref[0]
        xh = x.astype(jnp.bfloat16)
        q = jnp.dot(xh, wq_ref[...], preferred_element_type=jnp.float32)
        k = jnp.dot(x, wk_ref[...], preferred_element_type=jnp.float32)
        v = jnp.dot(xh, wv_ref[...], preferred_element_type=jnp.float32)
        q_ref[pl.ds(t * bs, bs), :] = q.astype(jnp.bfloat16)

        dot = jnp.sum(x * we_ref[...], axis=-1, keepdims=True) + be_ref[0, 0]
        w = base * 2.0 * jax.nn.sigmoid(dot)

        kn = k / (jnp.sqrt(jnp.sum(k * k, axis=-1, keepdims=True)) + EPS)

        @pl.when(t == 0)
        def _():
            cn_ref[...] = kn[:K]
            sk_ref[...] = jnp.zeros_like(sk_ref)
            sv_ref[...] = jnp.zeros_like(sv_ref)
            cnt_ref[...] = jnp.zeros_like(cnt_ref)

        sims = lax.dot_general(kn, cn_ref[...], (((1,), (1,)), ((), ())),
                               preferred_element_type=jnp.float32)
        m = jnp.max(sims, axis=-1, keepdims=True)
        j = lax.broadcasted_iota(jnp.int32, sims.shape, 1)
        assign = jnp.min(jnp.where(sims >= m, j, K), axis=-1, keepdims=True)
        p = (j == assign).astype(jnp.float32)

        sk_ref[...] += lax.dot_general(p, k * w, (((0,), (0,)), ((), ())),
                                       preferred_element_type=jnp.float32)
        sv_ref[...] += lax.dot_general(p, v * w, (((0,), (0,)), ((), ())),
                                       preferred_element_type=jnp.float32)
        cnt_ref[0:1, :] += jnp.sum(p * w, axis=0, keepdims=True)

    @pl.when(t >= ns)
    def _phase1():
        inv = 1.0 / (cnt_ref[0:1, :] + EPS)
        scale = inv / jnp.sqrt(jnp.float32(dh))
        q = q_ref[pl.ds((t - ns) * bs, bs), :]
        for h in range(H):
            lo = h * dh
            qh = q[:, lo:lo + dh]
            skh = sk_ref[:, lo:lo + dh].astype(jnp.bfloat16)
            svh = sv_ref[:, lo:lo + dh].astype(jnp.bfloat16)
            scores = lax.dot_general(qh, skh, (((1,), (1,)), ((), ())),
                                     preferred_element_type=jnp.float32) * scale
            mx = jnp.max(scores, axis=-1, keepdims=True)
            e = jnp.exp(scores - mx)
            attn = (e / jnp.sum(e, axis=-1, keepdims=True)) * inv
            oh = jnp.dot(attn.astype(jnp.bfloat16), svh,
                         preferred_element_type=jnp.float32)
            acc_ref[:, lo:lo + dh] = oh.astype(jnp.bfloat16)
        out_ref[0] = jnp.dot(acc_ref[...], wo_ref[...],
                             preferred_element_type=jnp.float32)


def kernel(x, W_q, W_k, W_v, W_o, w_e, b_e):
    B, S, D = x.shape
    K = MAX_STATES
    BS = 1024 if S % 1024 == 0 else (512 if S % 512 == 0 else S)
    ns = S // BS
    base = ENERGY_BUDGET / S

    we_row = w_e.reshape(1, D)
    be_11 = b_e.reshape(1, 1)
    wq_h = W_q.astype(jnp.bfloat16)
    wv_h = W_v.astype(jnp.bfloat16)
    wo_h = W_o.astype(jnp.bfloat16)

    full = lambda b, t: (0, 0)
    out = pl.pallas_call(
        functools.partial(_body, base=base, ns=ns, bs=BS),
        grid=(B, 2 * ns),
        in_specs=[
            pl.BlockSpec((1, BS, D),
                         lambda b, t: (b, jnp.minimum(t, ns - 1), 0)),
            pl.BlockSpec((D, D), full),
            pl.BlockSpec((D, D), full),
            pl.BlockSpec((D, D), full),
            pl.BlockSpec((D, D), full),
            pl.BlockSpec((1, D), full),
            pl.BlockSpec((1, 1), full),
        ],
        out_specs=pl.BlockSpec((1, BS, D),
                               lambda b, t: (b, jnp.maximum(t - ns, 0), 0)),
        out_shape=jax.ShapeDtypeStruct((B, S, D), jnp.float32),
        scratch_shapes=[
            pltpu.VMEM((S, D), jnp.bfloat16),
            pltpu.VMEM((K, D), jnp.float32),
            pltpu.VMEM((K, D), jnp.float32),
            pltpu.VMEM((8, K), jnp.float32),
            pltpu.VMEM((K, D), jnp.float32),
            pltpu.VMEM((BS, D), jnp.bfloat16),
        ],
        compiler_params=pltpu.CompilerParams(
            dimension_semantics=("arbitrary", "arbitrary")),
    )(x, wq_h, W_k, wv_h, wo_h, we_row, be_11)
    return out
